# Initial kernel scaffold; baseline (speedup 1.0000x reference)
import jax
import jax.numpy as jnp
from jax.experimental import pallas as pl


def kernel(inputs, lookup_table):
    raise NotImplementedError("write your pallas kernel here")


# speedup vs baseline: 1.0144x; 1.0144x over previous
import functools

import jax
import jax.numpy as jnp
from jax import lax
from jax.experimental import pallas as pl
from jax.experimental.pallas import tpu as pltpu
from jax.experimental.pallas import tpu_sc as plsc

NC, NS, L = 2, 16, 16
NW = NC * NS
GRP = 128
CH = 1024
N_GRP = CH // GRP


@functools.lru_cache(maxsize=None)
def _make(B, V, D):
  assert B % (NW * CH) == 0 and D % L == 0
  b_per_w = B // NW
  n_ch = b_per_w // CH
  scale = jnp.float32(D) ** 0.5
  mesh = plsc.VectorSubcoreMesh(core_axis_name="c", subcore_axis_name="s")

  @functools.partial(
      pl.kernel,
      out_type=jax.ShapeDtypeStruct((B, D), jnp.float32),
      mesh=mesh,
      scratch_types=[
          pltpu.VMEM((N_GRP, GRP), jnp.int32),
          pltpu.VMEM((CH, D), jnp.float32),
          pltpu.SemaphoreType.DMA,
      ],
      compiler_params=pltpu.CompilerParams(use_tc_tiling_on_sc=False),
  )
  def k(idx_hbm, table_hbm, out_hbm, idx_v, rows_v, sem):
    wid = lax.axis_index("s") * NC + lax.axis_index("c")
    base = wid * b_per_w

    @pl.loop(0, n_ch)
    def _chunk(g):
      off = base + g * CH
      pltpu.sync_copy(
          idx_hbm.at[pl.ds(pl.multiple_of(off // GRP, 8), N_GRP)], idx_v
      )
      cps = [
          pltpu.async_copy(
              table_hbm.at[idx_v.at[j]], rows_v.at[pl.ds(j * GRP, GRP)], sem
          )
          for j in range(N_GRP)
      ]
      for cp in cps:
        cp.wait()

      @plsc.parallel_loop(0, CH, unroll=8)
      def _scale(i):
        rows_v[i, pl.ds(0, L)] = rows_v[i, pl.ds(0, L)] * scale
        rows_v[i, pl.ds(L, L)] = rows_v[i, pl.ds(L, L)] * scale

      pltpu.sync_copy(rows_v, out_hbm.at[pl.ds(off, CH)])

  return k


@jax.jit
def kernel(inputs, lookup_table):
  B0, B1 = inputs.shape
  V, D = lookup_table.shape
  B = B0 * B1
  idx = inputs.reshape(B // GRP, GRP).astype(jnp.int32)
  out = _make(B, V, D)(idx, lookup_table)
  return out.reshape(B0, B1, D)


# speedup vs baseline: 1.0494x; 1.0344x over previous
import functools

import jax
import jax.numpy as jnp
from jax import lax
from jax.experimental import pallas as pl
from jax.experimental.pallas import tpu as pltpu
from jax.experimental.pallas import tpu_sc as plsc

NC, NS, L = 2, 16, 16
NW = NC * NS
GRP = 128
CH = 1024
N_GRP = CH // GRP
NBUF = 3


@functools.lru_cache(maxsize=None)
def _make(B, V, D):
  assert B % (NW * CH) == 0 and D % L == 0
  b_per_w = B // NW
  n_ch = b_per_w // CH
  assert n_ch >= NBUF
  grp_per_w = b_per_w // GRP
  scale = jnp.float32(D) ** 0.5
  mesh = plsc.VectorSubcoreMesh(core_axis_name="c", subcore_axis_name="s")

  @functools.partial(
      pl.kernel,
      out_type=jax.ShapeDtypeStruct((B, D), jnp.float32),
      mesh=mesh,
      scratch_types=[
          pltpu.VMEM((grp_per_w, GRP), jnp.int32),
          pltpu.VMEM((NBUF, CH, D), jnp.float32),
          pltpu.SemaphoreType.DMA((NBUF,)),
          pltpu.SemaphoreType.DMA((NBUF,)),
      ],
      compiler_params=pltpu.CompilerParams(use_tc_tiling_on_sc=False),
  )
  def k(idx_hbm, table_hbm, out_hbm, idx_v, rows, gsem, wsem):
    wid = lax.axis_index("s") * NC + lax.axis_index("c")
    base = wid * b_per_w

    pltpu.sync_copy(
        idx_hbm.at[pl.ds(pl.multiple_of(wid * grp_per_w, 8), grp_per_w)],
        idx_v,
    )

    def start_gather(g, buf):
      for j in range(N_GRP):
        pltpu.async_copy(
            table_hbm.at[idx_v.at[g * N_GRP + j]],
            rows.at[buf].at[pl.ds(j * GRP, GRP)],
            gsem.at[buf],
        )

    def wait_gather(buf):
      for j in range(N_GRP):
        pltpu.make_async_copy(
            table_hbm.at[idx_v.at[j]],
            rows.at[buf].at[pl.ds(j * GRP, GRP)],
            gsem.at[buf],
        ).wait()

    def out_slice(g):
      return out_hbm.at[pl.ds(base + g * CH, CH)]

    def wait_writeout(buf):
      pltpu.make_async_copy(rows.at[buf], out_slice(0), wsem.at[buf]).wait()

    for b in range(NBUF - 1):
      start_gather(b, b)

    @pl.loop(0, n_ch)
    def _chunk(g):
      buf = lax.rem(g, NBUF)
      wait_gather(buf)

      g2 = g + NBUF - 1
      nbuf = lax.rem(g2, NBUF)

      @pl.when(g2 < n_ch)
      def _():
        @pl.when(g >= 1)
        def _():
          wait_writeout(nbuf)

        start_gather(g2, nbuf)

      @plsc.parallel_loop(0, CH, unroll=8)
      def _scale(i):
        rows[buf, i, pl.ds(0, L)] = rows[buf, i, pl.ds(0, L)] * scale
        rows[buf, i, pl.ds(L, L)] = rows[buf, i, pl.ds(L, L)] * scale

      pltpu.async_copy(rows.at[buf], out_slice(g), wsem.at[buf])

    for c in range(n_ch - NBUF, n_ch):
      wait_writeout(c % NBUF)

  return k


@jax.jit
def kernel(inputs, lookup_table):
  B0, B1 = inputs.shape
  V, D = lookup_table.shape
  B = B0 * B1
  idx = inputs.reshape(B // GRP, GRP).astype(jnp.int32)
  out = _make(B, V, D)(idx, lookup_table)
  return out.reshape(B0, B1, D)
